# Initial kernel scaffold; baseline (speedup 1.0000x reference)
import jax
import jax.numpy as jnp
from jax.experimental import pallas as pl


def kernel(states, actions, params):
    raise NotImplementedError("write your pallas kernel here")


# speedup vs baseline: 39.9068x; 39.9068x over previous
import functools

import numpy as np
import jax
import jax.numpy as jnp
from jax.experimental import pallas as pl

_A = 10
_SD = 256
_NA = 32
_S = 32


@functools.lru_cache(maxsize=2)
def _coalition_constants(B):
    with jax.ensure_compile_time_eval():
        key = jax.random.key(42)
        keys = jax.random.split(key, B * _S)
        gc = jax.vmap(lambda k: jax.random.permutation(k, _A))(keys)
        gc = np.asarray(gc).reshape(B, _S, _A).astype(np.int32)
    pm = np.ascontiguousarray(np.transpose(gc, (1, 2, 0)).reshape(_S * _A, B))
    comp = np.take_along_axis(gc, gc, axis=2)
    C = (comp[..., None] == np.arange(_A)).astype(np.float32).mean(axis=1)
    C2 = np.ascontiguousarray(np.transpose(C, (1, 2, 0)).reshape(_A * _A, B))
    return pm, C2


def _mlp_body(states_ref, act_ref,
              wW1, wW2, wW3, wW4, wb1, wb2, wb3,
              fW1, fW2, fW3, fW4, fb1, fb2, fb3,
              gW1, gW2, gW3, gW4, gb1, gb2, gb3,
              ow, of, og):
    xs = states_ref[...]
    xa = act_ref[0]
    groups = (
        (wW1, wW2, wW3, wW4, wb1, wb2, wb3, ow),
        (fW1, fW2, fW3, fW4, fb1, fb2, fb3, of),
        (gW1, gW2, gW3, gW4, gb1, gb2, gb3, og),
    )
    for W1, W2, W3, W4, b1, b2, b3, out in groups:
        w1 = W1[0]
        h = jnp.dot(xs, w1[0:_SD, :], preferred_element_type=jnp.float32)
        h = h + jnp.dot(xa, w1[_SD:_SD + _NA, :],
                        preferred_element_type=jnp.float32)
        h = jnp.maximum(h + b1[0], 0.0)
        h = jnp.maximum(jnp.dot(h, W2[0], preferred_element_type=jnp.float32)
                        + b2[0], 0.0)
        h = jnp.maximum(jnp.dot(h, W3[0], preferred_element_type=jnp.float32)
                        + b3[0], 0.0)
        out[0, 0, :] = jnp.dot(h, W4[0], preferred_element_type=jnp.float32)[:, 0]


def _coal_body(pm_ref, C_ref, f_ref, g_ref, w_ref,
               shap_ref, opt_ref, winv_ref):
    B = f_ref.shape[-1]
    pm = pm_ref[...].reshape(_S, _A, B)
    f = f_ref[...]
    g = g_ref[...]
    fperm = jnp.zeros((_S, _A, B), jnp.float32)
    for a in range(_A):
        fperm = fperm + jnp.where(pm == a, f[a], 0.0)
    run = jnp.ones((_S, B), jnp.float32)
    Ps = [run]
    for c in range(1, _A):
        run = run * fperm[:, c - 1, :]
        Ps.append(run)
    prodsel = jnp.zeros((_S, _A, B), jnp.float32)
    for c in range(_A):
        prodsel = prodsel + jnp.where(pm == c, Ps[c][:, None, :], 0.0)
    term1 = jnp.mean(prodsel, axis=0)
    C3 = C_ref[...].reshape(_A, _A, B)
    t2f = jnp.sum(C3 * f[None, :, :], axis=1)
    t2g = jnp.sum(C3 * g[None, :, :], axis=1)
    shap_ref[...] = term1 + t2f - t2g
    opt_ref[...] = term1 + t2f
    winv_ref[...] = 1.0 / w_ref[...]


def kernel(states, actions, params):
    B = states.shape[0]
    f32 = jnp.float32
    pm_np, C_np = _coalition_constants(B)
    pm_c = jnp.asarray(pm_np)
    C_c = jnp.asarray(C_np)

    act_t = jnp.transpose(actions.astype(f32), (1, 0, 2))

    def prep(p):
        return (p["W1"], p["W2"], p["W3"],
                p["W4"],
                p["b1"][:, None, :], p["b2"][:, None, :], p["b3"][:, None, :])

    wp, fp, gp = prep(params["w"]), prep(params["f"]), prep(params["g"])

    wspec = [
        pl.BlockSpec((1, _SD + _NA, 256), lambda a: (a, 0, 0)),
        pl.BlockSpec((1, 256, 256), lambda a: (a, 0, 0)),
        pl.BlockSpec((1, 256, 256), lambda a: (a, 0, 0)),
        pl.BlockSpec((1, 256, 1), lambda a: (a, 0, 0)),
        pl.BlockSpec((1, 1, 256), lambda a: (a, 0, 0)),
        pl.BlockSpec((1, 1, 256), lambda a: (a, 0, 0)),
        pl.BlockSpec((1, 1, 256), lambda a: (a, 0, 0)),
    ]
    est = pl.pallas_call(
        _mlp_body,
        grid=(_A,),
        in_specs=[
            pl.BlockSpec((B, _SD), lambda a: (0, 0)),
            pl.BlockSpec((1, B, _NA), lambda a: (a, 0, 0)),
        ] + wspec * 3,
        out_specs=[pl.BlockSpec((1, 1, B), lambda a: (a, 0, 0))] * 3,
        out_shape=[jax.ShapeDtypeStruct((_A, 1, B), f32)] * 3,
    )(states, act_t, *wp, *fp, *gp)

    w_t = est[0].reshape(_A, B) + params["w"]["b4"]
    f_t = est[1].reshape(_A, B) + params["f"]["b4"]
    g_t = est[2].reshape(_A, B) + params["g"]["b4"]

    shap, opt, winv = pl.pallas_call(
        _coal_body,
        out_shape=[jax.ShapeDtypeStruct((_A, B), f32)] * 3,
    )(pm_c, C_c, f_t, g_t, w_t)

    def to_out(x):
        return jnp.transpose(x)[..., None]

    return to_out(shap), to_out(opt), to_out(winv), to_out(w_t)


# speedup vs baseline: 40.4080x; 1.0126x over previous
import functools

import numpy as np
import jax
import jax.numpy as jnp
from jax.experimental import pallas as pl
from jax.experimental.pallas import tpu as pltpu

_A = 10
_SD = 256
_NA = 32
_S = 32


@functools.lru_cache(maxsize=2)
def _coalition_constants(B):
    with jax.ensure_compile_time_eval():
        key = jax.random.key(42)
        keys = jax.random.split(key, B * _S)
        gc = jax.vmap(lambda k: jax.random.permutation(k, _A))(keys)
        gc = np.asarray(gc).reshape(B, _S, _A).astype(np.int32)
    pm = np.ascontiguousarray(np.transpose(gc, (1, 2, 0)).reshape(_S * _A, B))
    comp = np.take_along_axis(gc, gc, axis=2)
    C = (comp[..., None] == np.arange(_A)).astype(np.float32).mean(axis=1)
    C2 = np.ascontiguousarray(np.transpose(C, (1, 2, 0)).reshape(_A * _A, B))
    return pm, C2


def _fused_body(states_ref, act_ref, pm_ref, C_ref,
                wW1, wW2, wW3, wW4, wb1, wb2, wb3, wb4,
                fW1, fW2, fW3, fW4, fb1, fb2, fb3, fb4,
                gW1, gW2, gW3, gW4, gb1, gb2, gb3, gb4,
                shap_out, opt_out, winv_out, west_out, scr):
    a = pl.program_id(0)
    B = states_ref.shape[0]
    f32 = jnp.float32
    xs = states_ref[...]
    xa = act_ref[0]
    groups = (
        (wW1, wW2, wW3, wW4, wb1, wb2, wb3, wb4),
        (fW1, fW2, fW3, fW4, fb1, fb2, fb3, fb4),
        (gW1, gW2, gW3, gW4, gb1, gb2, gb3, gb4),
    )
    for m, (W1, W2, W3, W4, b1, b2, b3, b4) in enumerate(groups):
        w1 = W1[0]
        h = jnp.dot(xs, w1[0:_SD, :], preferred_element_type=f32)
        h = h + jnp.dot(xa, w1[_SD:_SD + _NA, :], preferred_element_type=f32)
        h = jnp.maximum(h + b1[0], 0.0)
        h = jnp.maximum(jnp.dot(h, W2[0], preferred_element_type=f32)
                        + b2[0], 0.0)
        h = jnp.maximum(jnp.dot(h, W3[0], preferred_element_type=f32)
                        + b3[0], 0.0)
        row = jax.lax.dot_general(W4[0], h, (((0,), (1,)), ((), ())),
                                  preferred_element_type=f32)
        scr[pl.ds(16 * m + a, 1), :] = row + b4[0]

    @pl.when(a == _A - 1)
    def _coalition():
        w = scr[0:_A]
        f = scr[16:16 + _A]
        g = scr[32:32 + _A]
        pm = pm_ref[...].reshape(_S, _A, B)
        fperm = jnp.zeros((_S, _A, B), f32)
        for i in range(_A):
            fperm = fperm + jnp.where(pm == i, f[i], 0.0)
        run = jnp.ones((_S, B), f32)
        Ps = [run]
        for c in range(1, _A):
            run = run * fperm[:, c - 1, :]
            Ps.append(run)
        prodsel = jnp.zeros((_S, _A, B), f32)
        for c in range(_A):
            prodsel = prodsel + jnp.where(pm == c, Ps[c][:, None, :], 0.0)
        term1 = jnp.mean(prodsel, axis=0)
        C3 = C_ref[...].reshape(_A, _A, B)
        t2f = jnp.sum(C3 * f[None, :, :], axis=1)
        t2g = jnp.sum(C3 * g[None, :, :], axis=1)
        shap_out[...] = jnp.transpose(term1 + t2f - t2g)
        opt_out[...] = jnp.transpose(term1 + t2f)
        winv_out[...] = jnp.transpose(1.0 / w)
        west_out[...] = jnp.transpose(w)


def kernel(states, actions, params):
    B = states.shape[0]
    f32 = jnp.float32
    pm_np, C_np = _coalition_constants(B)
    pm_c = jnp.asarray(pm_np)
    C_c = jnp.asarray(C_np)

    act_t = jnp.transpose(actions.astype(f32), (1, 0, 2))

    def prep(p):
        return (p["W1"], p["W2"], p["W3"], p["W4"],
                p["b1"][:, None, :], p["b2"][:, None, :], p["b3"][:, None, :],
                jnp.broadcast_to(p["b4"][:, :, None], (_A, 1, B)))

    wp, fp, gp = prep(params["w"]), prep(params["f"]), prep(params["g"])

    wspec = [
        pl.BlockSpec((1, _SD + _NA, 256), lambda a: (a, 0, 0)),
        pl.BlockSpec((1, 256, 256), lambda a: (a, 0, 0)),
        pl.BlockSpec((1, 256, 256), lambda a: (a, 0, 0)),
        pl.BlockSpec((1, 256, 1), lambda a: (a, 0, 0)),
        pl.BlockSpec((1, 1, 256), lambda a: (a, 0, 0)),
        pl.BlockSpec((1, 1, 256), lambda a: (a, 0, 0)),
        pl.BlockSpec((1, 1, 256), lambda a: (a, 0, 0)),
        pl.BlockSpec((1, 1, B), lambda a: (a, 0, 0)),
    ]
    outs = pl.pallas_call(
        _fused_body,
        grid=(_A,),
        in_specs=[
            pl.BlockSpec((B, _SD), lambda a: (0, 0)),
            pl.BlockSpec((1, B, _NA), lambda a: (a, 0, 0)),
            pl.BlockSpec((_S * _A, B), lambda a: (0, 0)),
            pl.BlockSpec((_A * _A, B), lambda a: (0, 0)),
        ] + wspec * 3,
        out_specs=[pl.BlockSpec((B, _A), lambda a: (0, 0))] * 4,
        out_shape=[jax.ShapeDtypeStruct((B, _A), f32)] * 4,
        scratch_shapes=[pltpu.VMEM((48, B), f32)],
    )(states, act_t, pm_c, C_c, *wp, *fp, *gp)

    shap, opt, winv, west = outs
    return shap[..., None], opt[..., None], winv[..., None], west[..., None]
